# baseline (speedup 1.0000x reference)
import jax
import jax.numpy as jnp
from jax import lax
from jax.experimental import pallas as pl
from jax.experimental.pallas import tpu as pltpu
from jax.experimental.pallas import tpu_sc as plsc

_B, _S, _D = 4, 8192, 1024
_M = 2048
_NC, _NS = 2, 16
_NW = _NC * _NS
_RPW = (_B * _M) // _NW
_K = 16
_NCHUNK = _RPW // _K
_NBUF = 4
_P = 2
_NGROUP = _NCHUNK // _NBUF
_WPB = _M // _RPW


def _body(x_hbm, idx_hbm, out_hbm, idx_v,
          buf0, buf1, buf2, buf3,
          gsem0, gsem1, gsem2, gsem3, ssem0, ssem1, ssem2, ssem3):
    wid = lax.axis_index("s") * _NC + lax.axis_index("c")
    base = wid * _RPW
    off = (wid // _WPB) * _S
    pltpu.sync_copy(idx_hbm.at[pl.ds(base, _RPW)], idx_v)
    bufs = (buf0, buf1, buf2, buf3)
    gsems = (gsem0, gsem1, gsem2, gsem3)
    ssems = (ssem0, ssem1, ssem2, ssem3)

    def start_gather(c, s):
        iv = idx_v[pl.ds(c * _K, _K)] + off
        pltpu.async_copy(x_hbm.at[iv], bufs[s], gsems[s])

    def wait_gather(s):
        pltpu.make_async_copy(x_hbm.at[pl.ds(0, _K)], bufs[s], gsems[s]).wait()

    def start_store(c, s):
        pltpu.async_copy(bufs[s], out_hbm.at[pl.ds(base + c * _K, _K)], ssems[s])

    def wait_store(c, s):
        pltpu.make_async_copy(bufs[s], out_hbm.at[pl.ds(base + c * _K, _K)],
                              ssems[s]).wait()

    for c0 in range(_P):
        start_gather(c0, c0 % _NBUF)

    def group(g, carry):
        for s in range(_NBUF):
            c = g * _NBUF + s
            n = c + _P
            sn = (s + _P) % _NBUF
            @pl.when(n < _NCHUNK)
            def _():
                @pl.when(n >= _NBUF)
                def _():
                    wait_store(n - _NBUF, sn)
                start_gather(n, sn)
            wait_gather(s)
            start_store(c, s)
        return carry

    lax.fori_loop(0, _NGROUP, group, 0)
    for m in range(_NCHUNK - _NBUF, _NCHUNK):
        wait_store(m, m % _NBUF)


@jax.jit
def _gather_flat(xf, idxf):
    mesh = plsc.VectorSubcoreMesh(core_axis_name="c", subcore_axis_name="s")
    f = pl.kernel(
        _body,
        mesh=mesh,
        out_type=jax.ShapeDtypeStruct((_B * _M, _D), jnp.float32),
        scratch_types=[
            pltpu.VMEM((_RPW,), jnp.int32),
            pltpu.VMEM((_K, _D), jnp.float32),
            pltpu.VMEM((_K, _D), jnp.float32),
            pltpu.VMEM((_K, _D), jnp.float32),
            pltpu.VMEM((_K, _D), jnp.float32),
            pltpu.SemaphoreType.DMA,
            pltpu.SemaphoreType.DMA,
            pltpu.SemaphoreType.DMA,
            pltpu.SemaphoreType.DMA,
            pltpu.SemaphoreType.DMA,
            pltpu.SemaphoreType.DMA,
            pltpu.SemaphoreType.DMA,
            pltpu.SemaphoreType.DMA,
        ],
    )
    return f(xf, idxf)


def kernel(x, indices_to_keep):
    idxf = indices_to_keep.astype(jnp.int32).reshape(_B * _M)
    xf = x.reshape(_B * _S, _D)
    out = _gather_flat(xf, idxf)
    return out.reshape(_B, _M, _D)
